# baseline (speedup 1.0000x reference)
import functools

import jax
import jax.numpy as jnp
from jax import lax
from jax.experimental import pallas as pl
from jax.experimental.pallas import tpu as pltpu
from jax.experimental.pallas import tpu_sc as plsc

L = 16


def _softplus_b05(x):
    bx = 0.5 * x
    safe = jnp.where(bx <= 14.0, bx, 0.0)
    return jnp.where(bx <= 14.0, 2.0 * jnp.log1p(jnp.exp(safe)), x)


def _dense_body(rbf_ref, ef_ref, w1_ref, b1_ref, w2_ref, b2_ref, w3_ref,
                b3_ref, hef_out):
    dn = (((1,), (1,)), ((), ()))
    x = lax.dot_general(rbf_ref[...], w1_ref[...], dn,
                        preferred_element_type=jnp.float32) + b1_ref[...]
    x = _softplus_b05(x)
    h = lax.dot_general(x, w2_ref[...], dn,
                        preferred_element_type=jnp.float32) + b2_ref[...]
    ef = lax.dot_general(ef_ref[...], w3_ref[...], dn,
                         preferred_element_type=jnp.float32) + b3_ref[...]
    hw = lax.bitcast_convert_type(h.astype(jnp.bfloat16), jnp.uint16)
    ew = lax.bitcast_convert_type(ef.astype(jnp.bfloat16), jnp.uint16)
    hef_out[...] = hw.astype(jnp.uint32) | (ew.astype(jnp.uint32) << 16)


@functools.lru_cache(maxsize=None)
def _make_dense(E, D, R):
    EB = 3200
    assert E % EB == 0
    grid = (E // EB,)
    blk = lambda shape: pl.BlockSpec(shape, lambda i: (0, 0))
    return pl.pallas_call(
        _dense_body,
        grid=grid,
        in_specs=[
            pl.BlockSpec((EB, R), lambda i: (i, 0)),
            pl.BlockSpec((EB, D), lambda i: (i, 0)),
            blk((D, R)), blk((1, D)),
            blk((D, D)), blk((1, D)),
            blk((D, D)), blk((1, D)),
        ],
        out_specs=[pl.BlockSpec((EB, D), lambda i: (i, 0))],
        out_shape=[jax.ShapeDtypeStruct((E, D), jnp.uint32)],
    )


@functools.lru_cache(maxsize=None)
def _make_sc(N, E, D):
    NS = 16
    CH = 80
    BKC = 25
    BKE = BKC * CH
    ESC = E // NS
    NBLK = ESC // BKE
    assert ESC * NS == E and NBLK * BKE == ESC
    HALF = N // 2
    assert HALF * 2 == N
    RPT = (HALF // NS // 8 + 2) * 8
    HP = RPT * NS
    assert HP >= HALF + 128
    NV = D // L
    assert NV * L == D
    assert 16 * (3 * CH * D + 2 * CH * D + 2 * BKE + 2 * CH) + HP * D \
        <= 2_097_000

    mesh = plsc.VectorSubcoreMesh(core_axis_name="c", subcore_axis_name="s")

    @functools.partial(
        pl.kernel,
        mesh=mesh,
        compiler_params=pltpu.CompilerParams(use_tc_tiling_on_sc=False,
                                             needs_layout_passes=False),
        out_type=jax.ShapeDtypeStruct((2, HP, D), jnp.float32),
        scratch_types=[
            pltpu.VMEM((BKE,), jnp.int32),
            pltpu.VMEM((BKE,), jnp.int32),
            [pltpu.VMEM((CH,), jnp.int32)] * 2,
            [pltpu.VMEM((CH, D), jnp.uint32)] * 2,
            [pltpu.VMEM((CH, D), jnp.float32)] * 3,
            pltpu.VMEM_SHARED((HP, D), jnp.float32),
            [pltpu.SemaphoreType.DMA] * 3,
            [pltpu.SemaphoreType.DMA] * 2,
            [pltpu.SemaphoreType.DMA] * 2,
        ],
    )
    def sc_kernel(nf_hbm, hef_hbm, src_hbm, dst_hbm, out_hbm,
                  srcblk, dstblk, idxv, hefv, nfv, acc,
                  sem_nf, sem_hef, sem_sc):
        cid = lax.axis_index("c")
        sid = lax.axis_index("s")
        base_node = cid * HALF
        tile_rows = sid * RPT
        tile_edges = sid * ESC

        @plsc.parallel_loop(0, CH, step=1, unroll=8)
        def _zrow(r):
            for j in range(NV):
                nfv[0][r, pl.ds(j * L, L)] = jnp.zeros((L,), jnp.float32)
        nfull, rem = divmod(RPT, CH)
        for k in range(nfull):
            pltpu.sync_copy(nfv[0], acc.at[pl.ds(tile_rows + k * CH, CH)])
        if rem:
            pltpu.sync_copy(nfv[0].at[pl.ds(0, rem)],
                            acc.at[pl.ds(tile_rows + nfull * CH, rem)])
        plsc.subcore_barrier()

        def _gather(j):
            return pltpu.async_copy(
                nf_hbm.at[srcblk.at[pl.ds(j * CH, CH)]], nfv[j % 3],
                sem_nf[j % 3])

        def _hef(blk_edges, j):
            return pltpu.async_copy(
                hef_hbm.at[pl.ds(blk_edges + j * CH, CH)], hefv[j % 2],
                sem_hef[j % 2])

        def block(i, _):
            blk_edges = tile_edges + i * BKE
            pltpu.sync_copy(src_hbm.at[pl.ds(blk_edges, BKE)], srcblk)
            pltpu.sync_copy(dst_hbm.at[pl.ds(blk_edges, BKE)], dstblk)
            gathers = {0: _gather(0), 1: _gather(1)}
            hefs = {0: _hef(blk_edges, 0)}
            scats = {}
            for j in range(BKC):
                bn, b2 = j % 3, j % 2
                if j - 1 in scats:
                    scats.pop(j - 1).wait()
                if j + 2 < BKC:
                    gathers[j + 2] = _gather(j + 2)
                if j + 1 < BKC:
                    hefs[j + 1] = _hef(blk_edges, j + 1)
                lane = lax.iota(jnp.int32, L)
                for k in range(CH // L):
                    d = dstblk[pl.ds(j * CH + k * L, L)]
                    loc = d - base_node
                    ok = (loc >= 0) & (loc < HALF)
                    dummy = HALF + ((sid * 40 + k * L + lane) & 127)
                    idxv[b2][pl.ds(k * L, L)] = jnp.where(ok, loc, dummy)
                gathers.pop(j).wait()
                hefs.pop(j).wait()

                @plsc.parallel_loop(0, CH, step=1, unroll=8)
                def row(r, bn=bn, b2=b2):
                    for t in range(NV):
                        s = pl.ds(t * L, L)
                        he = plsc.bitcast(hefv[b2][r, s], jnp.bfloat16)
                        h, e = plsc.unpack(
                            he, format=plsc.PackFormat.INTERLEAVED,
                            preferred_element_type=jnp.float32)
                        nfv[bn][r, s] = nfv[bn][r, s] * h + e
                scats[j] = pltpu.async_copy(nfv[bn], acc.at[idxv[b2]],
                                            sem_sc[b2], add=True)
            for j in sorted(scats):
                scats.pop(j).wait()
            return 0

        lax.fori_loop(0, NBLK, block, 0)
        plsc.subcore_barrier()
        pltpu.sync_copy(acc.at[pl.ds(tile_rows, RPT)],
                        out_hbm.at[cid, pl.ds(tile_rows, RPT)])

    return sc_kernel, HALF


def kernel(node_feat, rbf, edge_f, edge_index, W1, b1, W2, b2, W3, b3):
    N, D = node_feat.shape
    E, R = rbf.shape
    dense = _make_dense(E, D, R)
    (hef,) = dense(rbf, edge_f, W1, b1.reshape(1, D), W2, b2.reshape(1, D),
                   W3, b3.reshape(1, D))
    sc, HALF = _make_sc(N, E, D)
    out2 = sc(node_feat, hef, edge_index[0], edge_index[1])
    return jnp.concatenate([out2[0, :HALF], out2[1, :HALF]], axis=0)


# speedup vs baseline: 1.0346x; 1.0346x over previous
import functools

import jax
import jax.numpy as jnp
from jax import lax
from jax.experimental import pallas as pl
from jax.experimental.pallas import tpu as pltpu
from jax.experimental.pallas import tpu_sc as plsc

L = 16


def _softplus_b05(x):
    bx = 0.5 * x
    safe = jnp.where(bx <= 14.0, bx, 0.0)
    return jnp.where(bx <= 14.0, 2.0 * jnp.log1p(jnp.exp(safe)), x)


def _dense_body(rbf_ref, ef_ref, w1_ref, b1_ref, w2_ref, b2_ref, w3_ref,
                b3_ref, hef_out):
    dn = (((1,), (1,)), ((), ()))
    x = lax.dot_general(rbf_ref[...], w1_ref[...], dn,
                        preferred_element_type=jnp.float32) + b1_ref[...]
    x = _softplus_b05(x)
    h = lax.dot_general(x, w2_ref[...], dn,
                        preferred_element_type=jnp.float32) + b2_ref[...]
    ef = lax.dot_general(ef_ref[...], w3_ref[...], dn,
                         preferred_element_type=jnp.float32) + b3_ref[...]
    hw = lax.bitcast_convert_type(h.astype(jnp.bfloat16), jnp.uint16)
    ew = lax.bitcast_convert_type(ef.astype(jnp.bfloat16), jnp.uint16)
    hef_out[...] = hw.astype(jnp.uint32) | (ew.astype(jnp.uint32) << 16)


@functools.lru_cache(maxsize=None)
def _make_dense(E, D, R):
    EB = 3200
    assert E % EB == 0
    grid = (E // EB,)
    blk = lambda shape: pl.BlockSpec(shape, lambda i: (0, 0))
    return pl.pallas_call(
        _dense_body,
        grid=grid,
        in_specs=[
            pl.BlockSpec((EB, R), lambda i: (i, 0)),
            pl.BlockSpec((EB, D), lambda i: (i, 0)),
            blk((D, R)), blk((1, D)),
            blk((D, D)), blk((1, D)),
            blk((D, D)), blk((1, D)),
        ],
        out_specs=[pl.BlockSpec((EB, D), lambda i: (i, 0))],
        out_shape=[jax.ShapeDtypeStruct((E, D), jnp.uint32)],
    )


def _packnf_body(nf_ref, out_ref):
    x = nf_ref[...]
    lo = lax.bitcast_convert_type(x[:, :32].astype(jnp.bfloat16), jnp.uint16)
    hi = lax.bitcast_convert_type(x[:, 32:].astype(jnp.bfloat16), jnp.uint16)
    out_ref[...] = lo.astype(jnp.uint32) | (hi.astype(jnp.uint32) << 16)


@functools.lru_cache(maxsize=None)
def _make_packnf(N, D):
    NB = 2000
    assert N % NB == 0
    return pl.pallas_call(
        _packnf_body,
        grid=(N // NB,),
        in_specs=[pl.BlockSpec((NB, D), lambda i: (i, 0))],
        out_specs=[pl.BlockSpec((NB, D // 2), lambda i: (i, 0))],
        out_shape=[jax.ShapeDtypeStruct((N, D // 2), jnp.uint32)],
    )


@functools.lru_cache(maxsize=None)
def _make_sc(N, E, D):
    NS = 16
    CH = 80
    BKC = 25
    BKE = BKC * CH
    ESC = E // NS
    NBLK = ESC // BKE
    assert ESC * NS == E and NBLK * BKE == ESC
    HALF = N // 2
    assert HALF * 2 == N
    RPT = (HALF // NS // 8 + 2) * 8
    HP = RPT * NS
    assert HP >= HALF + 128
    NV = D // L
    assert NV * L == D
    assert 16 * (CH * D + 2 * CH * D + 2 * CH * D + 2 * BKE + 2 * CH) \
        + HP * D <= 2_097_000

    mesh = plsc.VectorSubcoreMesh(core_axis_name="c", subcore_axis_name="s")

    @functools.partial(
        pl.kernel,
        mesh=mesh,
        compiler_params=pltpu.CompilerParams(use_tc_tiling_on_sc=False,
                                             needs_layout_passes=False),
        out_type=jax.ShapeDtypeStruct((2, HP, D), jnp.float32),
        scratch_types=[
            pltpu.VMEM((BKE,), jnp.int32),
            pltpu.VMEM((BKE,), jnp.int32),
            [pltpu.VMEM((CH,), jnp.int32)] * 2,
            [pltpu.VMEM((CH, D), jnp.uint32)] * 2,
            [pltpu.VMEM((CH, D // 2), jnp.uint32)] * 2,
            [pltpu.VMEM((CH, D), jnp.float32)] * 2,
            pltpu.VMEM_SHARED((HP, D), jnp.float32),
            [pltpu.SemaphoreType.DMA] * 2,
            [pltpu.SemaphoreType.DMA] * 2,
            [pltpu.SemaphoreType.DMA] * 2,
        ],
    )
    def sc_kernel(nf_hbm, hef_hbm, src_hbm, dst_hbm, out_hbm,
                  srcblk, dstblk, idxv, hefv, nfv, msgv, acc,
                  sem_nf, sem_hef, sem_sc):
        cid = lax.axis_index("c")
        sid = lax.axis_index("s")
        base_node = cid * HALF
        tile_rows = sid * RPT
        tile_edges = sid * ESC

        @plsc.parallel_loop(0, CH, step=1, unroll=8)
        def _zrow(r):
            for j in range(NV):
                msgv[0][r, pl.ds(j * L, L)] = jnp.zeros((L,), jnp.float32)
        nfull, rem = divmod(RPT, CH)
        for k in range(nfull):
            pltpu.sync_copy(msgv[0], acc.at[pl.ds(tile_rows + k * CH, CH)])
        if rem:
            pltpu.sync_copy(msgv[0].at[pl.ds(0, rem)],
                            acc.at[pl.ds(tile_rows + nfull * CH, rem)])
        plsc.subcore_barrier()

        def _gather(j):
            return pltpu.async_copy(
                nf_hbm.at[srcblk.at[pl.ds(j * CH, CH)]], nfv[j % 2],
                sem_nf[j % 2])

        def _hef(blk_edges, j):
            return pltpu.async_copy(
                hef_hbm.at[pl.ds(blk_edges + j * CH, CH)], hefv[j % 2],
                sem_hef[j % 2])

        def block(i, _):
            blk_edges = tile_edges + i * BKE
            pltpu.sync_copy(src_hbm.at[pl.ds(blk_edges, BKE)], srcblk)
            pltpu.sync_copy(dst_hbm.at[pl.ds(blk_edges, BKE)], dstblk)
            gathers = {0: _gather(0)}
            hefs = {0: _hef(blk_edges, 0)}
            scats = {}
            for j in range(BKC):
                b2 = j % 2
                if j - 2 in scats:
                    scats.pop(j - 2).wait()
                if j + 1 < BKC:
                    gathers[j + 1] = _gather(j + 1)
                    hefs[j + 1] = _hef(blk_edges, j + 1)
                lane = lax.iota(jnp.int32, L)
                for k in range(CH // L):
                    d = dstblk[pl.ds(j * CH + k * L, L)]
                    loc = d - base_node
                    ok = (loc >= 0) & (loc < HALF)
                    dummy = HALF + ((sid * 40 + k * L + lane) & 127)
                    idxv[b2][pl.ds(k * L, L)] = jnp.where(ok, loc, dummy)
                gathers.pop(j).wait()
                hefs.pop(j).wait()

                @plsc.parallel_loop(0, CH, step=1, unroll=8)
                def row(r, b2=b2):
                    unpk = lambda w: plsc.unpack(
                        plsc.bitcast(w, jnp.bfloat16),
                        format=plsc.PackFormat.INTERLEAVED,
                        preferred_element_type=jnp.float32)
                    a0, c0 = unpk(nfv[b2][r, pl.ds(0, L)])
                    a1, c1 = unpk(nfv[b2][r, pl.ds(L, L)])
                    nfb = (a0, a1, c0, c1)
                    for t in range(NV):
                        s = pl.ds(t * L, L)
                        h, e = unpk(hefv[b2][r, s])
                        msgv[b2][r, s] = nfb[t] * h + e
                scats[j] = pltpu.async_copy(msgv[b2], acc.at[idxv[b2]],
                                            sem_sc[b2], add=True)
            for j in sorted(scats):
                scats.pop(j).wait()
            return 0

        lax.fori_loop(0, NBLK, block, 0)
        plsc.subcore_barrier()
        pltpu.sync_copy(acc.at[pl.ds(tile_rows, RPT)],
                        out_hbm.at[cid, pl.ds(tile_rows, RPT)])

    return sc_kernel, HALF


def kernel(node_feat, rbf, edge_f, edge_index, W1, b1, W2, b2, W3, b3):
    N, D = node_feat.shape
    E, R = rbf.shape
    dense = _make_dense(E, D, R)
    (hef,) = dense(rbf, edge_f, W1, b1.reshape(1, D), W2, b2.reshape(1, D),
                   W3, b3.reshape(1, D))
    (nfp,) = _make_packnf(N, D)(node_feat)
    sc, HALF = _make_sc(N, E, D)
    out2 = sc(nfp, hef, edge_index[0], edge_index[1])
    return jnp.concatenate([out2[0, :HALF], out2[1, :HALF]], axis=0)


# speedup vs baseline: 1.0406x; 1.0058x over previous
import functools

import jax
import jax.numpy as jnp
from jax import lax
from jax.experimental import pallas as pl
from jax.experimental.pallas import tpu as pltpu
from jax.experimental.pallas import tpu_sc as plsc

L = 16


def _softplus_b05(x):
    bx = 0.5 * x
    safe = jnp.where(bx <= 14.0, bx, 0.0)
    return jnp.where(bx <= 14.0, 2.0 * jnp.log1p(jnp.exp(safe)), x)


def _dense_body(rbf_ref, ef_ref, w1_ref, b1_ref, w2_ref, b2_ref, w3_ref,
                b3_ref, hef_out):
    dn = (((1,), (1,)), ((), ()))
    x = lax.dot_general(rbf_ref[...], w1_ref[...], dn,
                        preferred_element_type=jnp.float32) + b1_ref[...]
    x = _softplus_b05(x)
    h = lax.dot_general(x, w2_ref[...], dn,
                        preferred_element_type=jnp.float32) + b2_ref[...]
    ef = lax.dot_general(ef_ref[...], w3_ref[...], dn,
                         preferred_element_type=jnp.float32) + b3_ref[...]
    hw = lax.bitcast_convert_type(h.astype(jnp.bfloat16), jnp.uint16)
    ew = lax.bitcast_convert_type(ef.astype(jnp.bfloat16), jnp.uint16)
    hef_out[...] = hw.astype(jnp.uint32) | (ew.astype(jnp.uint32) << 16)


@functools.lru_cache(maxsize=None)
def _make_dense(E, D, R):
    EB = 3200
    assert E % EB == 0
    grid = (E // EB,)
    blk = lambda shape: pl.BlockSpec(shape, lambda i: (0, 0))
    return pl.pallas_call(
        _dense_body,
        grid=grid,
        in_specs=[
            pl.BlockSpec((EB, R), lambda i: (i, 0)),
            pl.BlockSpec((EB, D), lambda i: (i, 0)),
            blk((D, R)), blk((1, D)),
            blk((D, D)), blk((1, D)),
            blk((D, D)), blk((1, D)),
        ],
        out_specs=[pl.BlockSpec((EB, D), lambda i: (i, 0))],
        out_shape=[jax.ShapeDtypeStruct((E, D), jnp.uint32)],
    )


def _packnf_body(nf_ref, out_ref):
    x = nf_ref[...]
    lo = lax.bitcast_convert_type(x[:, :32].astype(jnp.bfloat16), jnp.uint16)
    hi = lax.bitcast_convert_type(x[:, 32:].astype(jnp.bfloat16), jnp.uint16)
    out_ref[...] = lo.astype(jnp.uint32) | (hi.astype(jnp.uint32) << 16)


@functools.lru_cache(maxsize=None)
def _make_packnf(N, D):
    NB = 2000
    assert N % NB == 0
    return pl.pallas_call(
        _packnf_body,
        grid=(N // NB,),
        in_specs=[pl.BlockSpec((NB, D), lambda i: (i, 0))],
        out_specs=[pl.BlockSpec((NB, D // 2), lambda i: (i, 0))],
        out_shape=[jax.ShapeDtypeStruct((N, D // 2), jnp.uint32)],
    )


@functools.lru_cache(maxsize=None)
def _make_sc(N, E, D):
    NS = 16
    CH = 80
    BKC = 25
    BKE = BKC * CH
    ESC = E // NS
    NBLK = ESC // BKE
    assert ESC * NS == E and NBLK * BKE == ESC
    HALF = N // 2
    assert HALF * 2 == N
    RPT = (HALF // NS // 8 + 1) * 8
    HP = RPT * NS
    assert HP >= HALF + 64
    NV = D // L
    assert NV * L == D
    assert 16 * (3 * CH * D // 2 + 2 * CH * D + 2 * CH * D + BKE + 3 * CH
                 + 2 * CH) + HP * D <= 2_097_100

    mesh = plsc.VectorSubcoreMesh(core_axis_name="c", subcore_axis_name="s")

    @functools.partial(
        pl.kernel,
        mesh=mesh,
        compiler_params=pltpu.CompilerParams(use_tc_tiling_on_sc=False,
                                             needs_layout_passes=False),
        out_type=jax.ShapeDtypeStruct((2, HP, D), jnp.float32),
        scratch_types=[
            pltpu.VMEM((BKE,), jnp.int32),
            [pltpu.VMEM((CH,), jnp.int32)] * 3,
            [pltpu.VMEM((CH,), jnp.int32)] * 2,
            [pltpu.VMEM((CH, D), jnp.uint32)] * 2,
            [pltpu.VMEM((CH, D // 2), jnp.uint32)] * 3,
            [pltpu.VMEM((CH, D), jnp.float32)] * 2,
            pltpu.VMEM_SHARED((HP, D), jnp.float32),
            [pltpu.SemaphoreType.DMA] * 3,
            [pltpu.SemaphoreType.DMA] * 3,
            [pltpu.SemaphoreType.DMA] * 2,
            [pltpu.SemaphoreType.DMA] * 2,
        ],
    )
    def sc_kernel(nf_hbm, hef_hbm, src_hbm, dst_hbm, out_hbm,
                  srcblk, dstv, idxv, hefv, nfv, msgv, acc,
                  sem_nf, sem_dst, sem_hef, sem_sc):
        cid = lax.axis_index("c")
        sid = lax.axis_index("s")
        base_node = cid * HALF
        tile_rows = sid * RPT
        tile_edges = sid * ESC

        @plsc.parallel_loop(0, CH, step=1, unroll=8)
        def _zrow(r):
            for j in range(NV):
                msgv[0][r, pl.ds(j * L, L)] = jnp.zeros((L,), jnp.float32)
        nfull, rem = divmod(RPT, CH)
        for k in range(nfull):
            pltpu.sync_copy(msgv[0], acc.at[pl.ds(tile_rows + k * CH, CH)])
        if rem:
            pltpu.sync_copy(msgv[0].at[pl.ds(0, rem)],
                            acc.at[pl.ds(tile_rows + nfull * CH, rem)])
        plsc.subcore_barrier()

        def _gather(j):
            return pltpu.async_copy(
                nf_hbm.at[srcblk.at[pl.ds(j * CH, CH)]], nfv[j % 3],
                sem_nf[j % 3])

        def _dst(blk_edges, j):
            return pltpu.async_copy(
                dst_hbm.at[pl.ds(blk_edges + j * CH, CH)], dstv[j % 3],
                sem_dst[j % 3])

        def _hef(blk_edges, j):
            return pltpu.async_copy(
                hef_hbm.at[pl.ds(blk_edges + j * CH, CH)], hefv[j % 2],
                sem_hef[j % 2])

        def block(i, _):
            blk_edges = tile_edges + i * BKE
            pltpu.sync_copy(src_hbm.at[pl.ds(blk_edges, BKE)], srcblk)
            gathers = {0: _gather(0), 1: _gather(1)}
            dsts = {0: _dst(blk_edges, 0), 1: _dst(blk_edges, 1),
                    2: _dst(blk_edges, 2)}
            hefs = {0: _hef(blk_edges, 0)}
            scats = {}
            for j in range(BKC):
                b2 = j % 2
                if j - 2 in scats:
                    scats.pop(j - 2).wait()
                if j + 2 < BKC:
                    gathers[j + 2] = _gather(j + 2)
                    if j + 3 < BKC:
                        dsts[j + 3] = _dst(blk_edges, j + 3)
                if j + 1 < BKC:
                    hefs[j + 1] = _hef(blk_edges, j + 1)
                dsts.pop(j).wait()
                lane = lax.iota(jnp.int32, L)
                for k in range(CH // L):
                    d = dstv[j % 3][pl.ds(k * L, L)]
                    loc = d - base_node
                    ok = (loc >= 0) & (loc < HALF)
                    dummy = HALF + ((sid * 24 + k * L + lane) & 63)
                    idxv[b2][pl.ds(k * L, L)] = jnp.where(ok, loc, dummy)
                gathers.pop(j).wait()
                hefs.pop(j).wait()

                @plsc.parallel_loop(0, CH, step=1, unroll=8)
                def row(r, b2=b2, b3=j % 3):
                    unpk = lambda w: plsc.unpack(
                        plsc.bitcast(w, jnp.bfloat16),
                        format=plsc.PackFormat.INTERLEAVED,
                        preferred_element_type=jnp.float32)
                    a0, c0 = unpk(nfv[b3][r, pl.ds(0, L)])
                    a1, c1 = unpk(nfv[b3][r, pl.ds(L, L)])
                    nfb = (a0, a1, c0, c1)
                    for t in range(NV):
                        s = pl.ds(t * L, L)
                        h, e = unpk(hefv[b2][r, s])
                        msgv[b2][r, s] = nfb[t] * h + e
                scats[j] = pltpu.async_copy(msgv[b2], acc.at[idxv[b2]],
                                            sem_sc[b2], add=True)
            for j in sorted(scats):
                scats.pop(j).wait()
            return 0

        lax.fori_loop(0, NBLK, block, 0)
        plsc.subcore_barrier()
        pltpu.sync_copy(acc.at[pl.ds(tile_rows, RPT)],
                        out_hbm.at[cid, pl.ds(tile_rows, RPT)])

    return sc_kernel, HALF


def kernel(node_feat, rbf, edge_f, edge_index, W1, b1, W2, b2, W3, b3):
    N, D = node_feat.shape
    E, R = rbf.shape
    dense = _make_dense(E, D, R)
    (hef,) = dense(rbf, edge_f, W1, b1.reshape(1, D), W2, b2.reshape(1, D),
                   W3, b3.reshape(1, D))
    (nfp,) = _make_packnf(N, D)(node_feat)
    sc, HALF = _make_sc(N, E, D)
    out2 = sc(nfp, hef, edge_index[0], edge_index[1])
    return jnp.concatenate([out2[0, :HALF], out2[1, :HALF]], axis=0)
